# baseline (device time: 84065 ns/iter reference)
import jax
import jax.numpy as jnp
from jax import lax
from jax.experimental import pallas as pl
from jax.experimental.pallas import tpu as pltpu

N_DEV = 4


def kernel(x, w_mat, scale_x, scale_w):
    m_per, k = x.shape
    _, n_per = w_mat.shape
    half = m_per // 2
    quarter = half // 2
    s = (scale_x * scale_w).reshape(1, 1)

    def body(x_hbm, w_hbm, s_ref, out_hbm,
             xv, wv, ov, cL, cR, cO,
             send_sems, recv_sems, load_sems, store_sems):
        my_pos = lax.axis_index("i")
        left = (my_pos - 1) % N_DEV
        right = (my_pos + 1) % N_DEV

        load_x = pltpu.make_async_copy(x_hbm, xv, load_sems.at[0])
        load_w = pltpu.make_async_copy(w_hbm, wv, load_sems.at[1])
        load_x.start()
        load_w.start()

        barrier_sem = pltpu.get_barrier_semaphore()
        for nbr in [left, right]:
            pl.semaphore_signal(
                barrier_sem, inc=1,
                device_id=(nbr,), device_id_type=pl.DeviceIdType.MESH,
            )
        pl.semaphore_wait(barrier_sem, 2)

        send_r = pltpu.make_async_remote_copy(
            src_ref=x_hbm, dst_ref=cL,
            send_sem=send_sems.at[0], recv_sem=recv_sems.at[0],
            device_id=(right,), device_id_type=pl.DeviceIdType.MESH,
        )
        send_l = pltpu.make_async_remote_copy(
            src_ref=x_hbm, dst_ref=cR,
            send_sem=send_sems.at[1], recv_sem=recv_sems.at[1],
            device_id=(left,), device_id_type=pl.DeviceIdType.MESH,
        )
        send_r.start()
        send_l.start()

        scale = s_ref[0, 0]

        def gemm_rows(chunk, slot, row_off, nrows):
            acc = lax.dot_general(
                chunk, wv[...],
                dimension_numbers=(((1,), (0,)), ((), ())),
                preferred_element_type=jnp.int32,
            )
            y = acc.astype(jnp.float32) * scale
            ov[slot, pl.ds(row_off, nrows), :] = y * jax.nn.sigmoid(y)

        def store_block(slot, origin, row_off, nrows, sem_idx):
            dma = pltpu.make_async_copy(
                ov.at[slot, pl.ds(row_off, nrows)],
                out_hbm.at[pl.ds(origin * m_per + row_off, nrows)],
                store_sems.at[sem_idx],
            )
            dma.start()
            return dma

        load_x.wait()
        load_w.wait()
        gemm_rows(xv[...], 0, 0, m_per)
        st0 = store_block(0, my_pos, 0, m_per, 0)

        def piece(buf, off, sem_idx, target):
            return pltpu.make_async_remote_copy(
                src_ref=buf.at[pl.ds(off, quarter)],
                dst_ref=cO.at[pl.ds(off, quarter)],
                send_sem=send_sems.at[sem_idx],
                recv_sem=recv_sems.at[sem_idx],
                device_id=(target,), device_id_type=pl.DeviceIdType.MESH,
            )

        send_r.wait_recv()
        fwd_r1 = piece(cL, 0, 2, right)
        fwd_r2 = piece(cL, quarter, 3, right)
        fwd_r1.start()
        fwd_r2.start()
        send_l.wait_recv()
        fwd_l1 = piece(cR, half, 4, left)
        fwd_l2 = piece(cR, half + quarter, 5, left)
        fwd_l1.start()
        fwd_l2.start()

        gemm_rows(cL[...], 1, 0, m_per)
        st1 = store_block(1, left, 0, m_per, 1)
        gemm_rows(cR[...], 3, 0, m_per)
        st3 = store_block(3, right, 0, m_per, 2)

        opp = (my_pos + 2) % N_DEV
        fwd_r1.wait_recv()
        gemm_rows(cO[pl.ds(0, quarter)], 2, 0, quarter)
        st2a = store_block(2, opp, 0, quarter, 3)
        fwd_l1.wait_recv()
        gemm_rows(cO[pl.ds(half, quarter)], 2, half, quarter)
        st2b = store_block(2, opp, half, quarter, 4)
        fwd_r2.wait_recv()
        gemm_rows(cO[pl.ds(quarter, quarter)], 2, quarter, quarter)
        st2c = store_block(2, opp, quarter, quarter, 5)
        fwd_l2.wait_recv()
        gemm_rows(cO[pl.ds(half + quarter, quarter)], 2, half + quarter,
                  quarter)
        st2d = store_block(2, opp, half + quarter, quarter, 6)

        send_r.wait_send()
        send_l.wait_send()
        for f in (fwd_r1, fwd_r2, fwd_l1, fwd_l2):
            f.wait_send()
        for st in (st0, st1, st3, st2a, st2b, st2c, st2d):
            st.wait()

    return pl.pallas_call(
        body,
        out_shape=jax.ShapeDtypeStruct((N_DEV * m_per, n_per), jnp.float32),
        in_specs=[
            pl.BlockSpec(memory_space=pl.ANY),
            pl.BlockSpec(memory_space=pl.ANY),
            pl.BlockSpec(memory_space=pltpu.SMEM),
        ],
        out_specs=pl.BlockSpec(memory_space=pl.ANY),
        scratch_shapes=[
            pltpu.VMEM((m_per, k), x.dtype),
            pltpu.VMEM((k, n_per), w_mat.dtype),
            pltpu.VMEM((N_DEV, m_per, n_per), jnp.float32),
            pltpu.VMEM((m_per, k), x.dtype),
            pltpu.VMEM((m_per, k), x.dtype),
            pltpu.VMEM((m_per, k), x.dtype),
            pltpu.SemaphoreType.DMA((6,)),
            pltpu.SemaphoreType.DMA((6,)),
            pltpu.SemaphoreType.DMA((2,)),
            pltpu.SemaphoreType.DMA((7,)),
        ],
        compiler_params=pltpu.CompilerParams(collective_id=0),
    )(x, w_mat, s)


# device time: 80772 ns/iter; 1.0408x vs baseline; 1.0408x over previous
import jax
import jax.numpy as jnp
from jax import lax
from jax.experimental import pallas as pl
from jax.experimental.pallas import tpu as pltpu

N_DEV = 4


def kernel(x, w_mat, scale_x, scale_w):
    m_per, k = x.shape
    _, n_per = w_mat.shape
    half = m_per // 2
    quarter = half // 2
    s = (scale_x * scale_w).reshape(1, 1)

    def body(x_hbm, w_hbm, s_ref, out_ref,
             xv, wv, cL, cR, cO,
             send_sems, recv_sems, load_sems):
        my_pos = lax.axis_index("i")
        left = (my_pos - 1) % N_DEV
        right = (my_pos + 1) % N_DEV

        load_x = pltpu.make_async_copy(x_hbm, xv, load_sems.at[0])
        load_w = pltpu.make_async_copy(w_hbm, wv, load_sems.at[1])
        load_x.start()
        load_w.start()

        barrier_sem = pltpu.get_barrier_semaphore()
        for nbr in [left, right]:
            pl.semaphore_signal(
                barrier_sem, inc=1,
                device_id=(nbr,), device_id_type=pl.DeviceIdType.MESH,
            )
        pl.semaphore_wait(barrier_sem, 2)

        send_r = pltpu.make_async_remote_copy(
            src_ref=x_hbm, dst_ref=cL,
            send_sem=send_sems.at[0], recv_sem=recv_sems.at[0],
            device_id=(right,), device_id_type=pl.DeviceIdType.MESH,
        )
        send_l = pltpu.make_async_remote_copy(
            src_ref=x_hbm, dst_ref=cR,
            send_sem=send_sems.at[1], recv_sem=recv_sems.at[1],
            device_id=(left,), device_id_type=pl.DeviceIdType.MESH,
        )
        send_r.start()
        send_l.start()

        scale = s_ref[0, 0]

        def gemm_rows(chunk, origin, row_off, nrows):
            acc = lax.dot_general(
                chunk, wv[...],
                dimension_numbers=(((1,), (0,)), ((), ())),
                preferred_element_type=jnp.int32,
            )
            y = acc.astype(jnp.float32) * scale
            out_ref[pl.ds(origin * m_per + row_off, nrows), :] = (
                y * jax.nn.sigmoid(y)
            )

        load_x.wait()
        load_w.wait()
        gemm_rows(xv[...], my_pos, 0, m_per)

        def piece(buf, off, sem_idx, target):
            return pltpu.make_async_remote_copy(
                src_ref=buf.at[pl.ds(off, quarter)],
                dst_ref=cO.at[pl.ds(off, quarter)],
                send_sem=send_sems.at[sem_idx],
                recv_sem=recv_sems.at[sem_idx],
                device_id=(target,), device_id_type=pl.DeviceIdType.MESH,
            )

        send_r.wait_recv()
        fwd_r1 = piece(cL, 0, 2, right)
        fwd_r2 = piece(cL, quarter, 3, right)
        fwd_r1.start()
        fwd_r2.start()
        send_l.wait_recv()
        fwd_l1 = piece(cR, half, 4, left)
        fwd_l2 = piece(cR, half + quarter, 5, left)
        fwd_l1.start()
        fwd_l2.start()

        gemm_rows(cL[...], left, 0, m_per)
        gemm_rows(cR[...], right, 0, m_per)

        opp = (my_pos + 2) % N_DEV
        fwd_r1.wait_recv()
        gemm_rows(cO[pl.ds(0, quarter)], opp, 0, quarter)
        fwd_l1.wait_recv()
        gemm_rows(cO[pl.ds(half, quarter)], opp, half, quarter)
        fwd_r2.wait_recv()
        gemm_rows(cO[pl.ds(quarter, quarter)], opp, quarter, quarter)
        fwd_l2.wait_recv()
        gemm_rows(cO[pl.ds(half + quarter, quarter)], opp, half + quarter,
                  quarter)

        send_r.wait_send()
        send_l.wait_send()
        for f in (fwd_r1, fwd_r2, fwd_l1, fwd_l2):
            f.wait_send()

    return pl.pallas_call(
        body,
        out_shape=jax.ShapeDtypeStruct((N_DEV * m_per, n_per), jnp.float32),
        in_specs=[
            pl.BlockSpec(memory_space=pltpu.MemorySpace.HBM),
            pl.BlockSpec(memory_space=pltpu.MemorySpace.HBM),
            pl.BlockSpec(memory_space=pltpu.SMEM),
        ],
        out_specs=pl.BlockSpec(memory_space=pltpu.VMEM),
        scratch_shapes=[
            pltpu.VMEM((m_per, k), x.dtype),
            pltpu.VMEM((k, n_per), w_mat.dtype),
            pltpu.VMEM((m_per, k), x.dtype),
            pltpu.VMEM((m_per, k), x.dtype),
            pltpu.VMEM((m_per, k), x.dtype),
            pltpu.SemaphoreType.DMA((6,)),
            pltpu.SemaphoreType.DMA((6,)),
            pltpu.SemaphoreType.DMA((2,)),
        ],
        compiler_params=pltpu.CompilerParams(collective_id=0),
    )(
        pltpu.with_memory_space_constraint(x, pltpu.MemorySpace.HBM),
        pltpu.with_memory_space_constraint(w_mat, pltpu.MemorySpace.HBM),
        s,
    )


# device time: 78186 ns/iter; 1.0752x vs baseline; 1.0331x over previous
import jax
import jax.numpy as jnp
from jax import lax
from jax.experimental import pallas as pl
from jax.experimental.pallas import tpu as pltpu

N_DEV = 4


def kernel(x, w_mat, scale_x, scale_w):
    m_per, k = x.shape
    _, n_per = w_mat.shape
    half = m_per // 2
    quarter = half // 2
    s = (scale_x * scale_w).reshape(1, 1)

    def body(x_hbm, w_hbm, s_ref, out_ref,
             xv, wv, cL, cR, cO,
             send_sems, recv_sems, load_sems):
        my_pos = lax.axis_index("i")
        left = (my_pos - 1) % N_DEV
        right = (my_pos + 1) % N_DEV

        load_x = pltpu.make_async_copy(x_hbm, xv, load_sems.at[0])
        load_w = pltpu.make_async_copy(w_hbm, wv, load_sems.at[1])
        load_x.start()
        load_w.start()

        barrier_sem = pltpu.get_barrier_semaphore()
        for nbr in [left, right]:
            pl.semaphore_signal(
                barrier_sem, inc=1,
                device_id=(nbr,), device_id_type=pl.DeviceIdType.MESH,
            )
        pl.semaphore_wait(barrier_sem, 2)

        def hop1(rows_off, sem_idx, target, dst):
            return pltpu.make_async_remote_copy(
                src_ref=x_hbm.at[pl.ds(rows_off, half)],
                dst_ref=dst.at[pl.ds(rows_off, half)],
                send_sem=send_sems.at[sem_idx],
                recv_sem=recv_sems.at[sem_idx],
                device_id=(target,), device_id_type=pl.DeviceIdType.MESH,
            )

        send_r_a = hop1(0, 0, right, cL)
        send_l_a = hop1(half, 1, left, cR)
        send_r_b = hop1(half, 2, right, cL)
        send_l_b = hop1(0, 3, left, cR)
        send_r_a.start()
        send_l_a.start()
        send_r_b.start()
        send_l_b.start()

        scale = s_ref[0, 0]

        def gemm_rows(chunk, origin, row_off, nrows):
            acc = lax.dot_general(
                chunk, wv[...],
                dimension_numbers=(((1,), (0,)), ((), ())),
                preferred_element_type=jnp.int32,
            )
            y = acc.astype(jnp.float32) * scale
            out_ref[pl.ds(origin * m_per + row_off, nrows), :] = (
                y * jax.nn.sigmoid(y)
            )

        load_x.wait()
        load_w.wait()
        gemm_rows(xv[...], my_pos, 0, m_per)

        def piece(buf, off, sem_idx, target):
            return pltpu.make_async_remote_copy(
                src_ref=buf.at[pl.ds(off, quarter)],
                dst_ref=cO.at[pl.ds(off, quarter)],
                send_sem=send_sems.at[sem_idx],
                recv_sem=recv_sems.at[sem_idx],
                device_id=(target,), device_id_type=pl.DeviceIdType.MESH,
            )

        send_r_a.wait_recv()
        fwd_r1 = piece(cL, 0, 4, right)
        fwd_r2 = piece(cL, quarter, 5, right)
        fwd_r1.start()
        fwd_r2.start()
        send_l_a.wait_recv()
        fwd_l1 = piece(cR, half, 6, left)
        fwd_l2 = piece(cR, half + quarter, 7, left)
        fwd_l1.start()
        fwd_l2.start()

        gemm_rows(cL[pl.ds(0, half)], left, 0, half)
        gemm_rows(cR[pl.ds(half, half)], right, half, half)
        send_r_b.wait_recv()
        gemm_rows(cL[pl.ds(half, half)], left, half, half)
        send_l_b.wait_recv()
        gemm_rows(cR[pl.ds(0, half)], right, 0, half)

        opp = (my_pos + 2) % N_DEV
        fwd_r1.wait_recv()
        gemm_rows(cO[pl.ds(0, quarter)], opp, 0, quarter)
        fwd_l1.wait_recv()
        gemm_rows(cO[pl.ds(half, quarter)], opp, half, quarter)
        fwd_r2.wait_recv()
        gemm_rows(cO[pl.ds(quarter, quarter)], opp, quarter, quarter)
        fwd_l2.wait_recv()
        gemm_rows(cO[pl.ds(half + quarter, quarter)], opp, half + quarter,
                  quarter)

        for snd in (send_r_a, send_l_a, send_r_b, send_l_b,
                    fwd_r1, fwd_r2, fwd_l1, fwd_l2):
            snd.wait_send()

    return pl.pallas_call(
        body,
        out_shape=jax.ShapeDtypeStruct((N_DEV * m_per, n_per), jnp.float32),
        in_specs=[
            pl.BlockSpec(memory_space=pltpu.MemorySpace.HBM),
            pl.BlockSpec(memory_space=pltpu.MemorySpace.HBM),
            pl.BlockSpec(memory_space=pltpu.SMEM),
        ],
        out_specs=pl.BlockSpec(memory_space=pltpu.VMEM),
        scratch_shapes=[
            pltpu.VMEM((m_per, k), x.dtype),
            pltpu.VMEM((k, n_per), w_mat.dtype),
            pltpu.VMEM((m_per, k), x.dtype),
            pltpu.VMEM((m_per, k), x.dtype),
            pltpu.VMEM((m_per, k), x.dtype),
            pltpu.SemaphoreType.DMA((8,)),
            pltpu.SemaphoreType.DMA((8,)),
            pltpu.SemaphoreType.DMA((2,)),
        ],
        compiler_params=pltpu.CompilerParams(collective_id=0),
    )(
        pltpu.with_memory_space_constraint(x, pltpu.MemorySpace.HBM),
        pltpu.with_memory_space_constraint(w_mat, pltpu.MemorySpace.HBM),
        s,
    )


# device time: 77068 ns/iter; 1.0908x vs baseline; 1.0145x over previous
import jax
import jax.numpy as jnp
from jax import lax
from jax.experimental import pallas as pl
from jax.experimental.pallas import tpu as pltpu

N_DEV = 4


def kernel(x, w_mat, scale_x, scale_w):
    m_per, k = x.shape
    _, n_per = w_mat.shape
    half = m_per // 2
    quarter = half // 2
    s = (scale_x * scale_w).reshape(1, 1)

    def body(x_hbm, w_hbm, s_ref, out_ref,
             xv, wv, cL, cR, cO,
             send_sems, recv_sems, load_sems):
        my_pos = lax.axis_index("i")
        left = (my_pos - 1) % N_DEV
        right = (my_pos + 1) % N_DEV

        load_x = pltpu.make_async_copy(x_hbm, xv, load_sems.at[0])
        load_w = pltpu.make_async_copy(w_hbm, wv, load_sems.at[1])
        load_x.start()
        load_w.start()

        barrier_sem = pltpu.get_barrier_semaphore()
        for nbr in [left, right]:
            pl.semaphore_signal(
                barrier_sem, inc=1,
                device_id=(nbr,), device_id_type=pl.DeviceIdType.MESH,
            )
        pl.semaphore_wait(barrier_sem, 2)

        def hop1(rows_off, sem_idx, target, dst):
            return pltpu.make_async_remote_copy(
                src_ref=x_hbm.at[pl.ds(rows_off, half)],
                dst_ref=dst.at[pl.ds(rows_off, half)],
                send_sem=send_sems.at[sem_idx],
                recv_sem=recv_sems.at[sem_idx],
                device_id=(target,), device_id_type=pl.DeviceIdType.MESH,
            )

        send_r_a = hop1(0, 0, right, cL)
        send_l_a = hop1(half, 1, left, cR)
        send_r_b = hop1(half, 2, right, cL)
        send_l_b = hop1(0, 3, left, cR)
        send_r_a.start()
        send_l_a.start()
        send_r_b.start()
        send_l_b.start()

        scale = s_ref[0, 0]

        def gemm_rows(chunk, origin, row_off, nrows):
            acc = lax.dot_general(
                chunk, wv[...],
                dimension_numbers=(((1,), (0,)), ((), ())),
                preferred_element_type=jnp.int32,
            )
            y = acc.astype(jnp.float32) * scale
            out_ref[pl.ds(origin * m_per + row_off, nrows), :] = (
                y * jax.nn.sigmoid(y)
            )

        load_x.wait()
        load_w.wait()
        gemm_rows(xv[...], my_pos, 0, m_per)

        eighth = quarter // 2

        def piece(buf, off, nrows, sem_idx, target):
            return pltpu.make_async_remote_copy(
                src_ref=buf.at[pl.ds(off, nrows)],
                dst_ref=cO.at[pl.ds(off, nrows)],
                send_sem=send_sems.at[sem_idx],
                recv_sem=recv_sems.at[sem_idx],
                device_id=(target,), device_id_type=pl.DeviceIdType.MESH,
            )

        send_r_a.wait_recv()
        fwd_r1 = piece(cL, 0, quarter, 4, right)
        fwd_r2 = piece(cL, quarter, eighth, 5, right)
        fwd_r3 = piece(cL, quarter + eighth, eighth, 6, right)
        fwd_r1.start()
        fwd_r2.start()
        fwd_r3.start()
        send_l_a.wait_recv()
        fwd_l1 = piece(cR, half, quarter, 7, left)
        fwd_l2 = piece(cR, half + quarter, eighth, 8, left)
        fwd_l3 = piece(cR, half + quarter + eighth, eighth, 9, left)
        fwd_l1.start()
        fwd_l2.start()
        fwd_l3.start()

        gemm_rows(cL[pl.ds(0, half)], left, 0, half)
        gemm_rows(cR[pl.ds(half, half)], right, half, half)
        send_r_b.wait_recv()
        gemm_rows(cL[pl.ds(half, half)], left, half, half)
        send_l_b.wait_recv()
        gemm_rows(cR[pl.ds(0, half)], right, 0, half)

        opp = (my_pos + 2) % N_DEV
        fwd_r1.wait_recv()
        gemm_rows(cO[pl.ds(0, quarter)], opp, 0, quarter)
        fwd_l1.wait_recv()
        gemm_rows(cO[pl.ds(half, quarter)], opp, half, quarter)
        fwd_r2.wait_recv()
        gemm_rows(cO[pl.ds(quarter, eighth)], opp, quarter, eighth)
        fwd_l2.wait_recv()
        gemm_rows(cO[pl.ds(half + quarter, eighth)], opp, half + quarter,
                  eighth)
        fwd_r3.wait_recv()
        gemm_rows(cO[pl.ds(quarter + eighth, eighth)], opp,
                  quarter + eighth, eighth)
        fwd_l3.wait_recv()
        gemm_rows(cO[pl.ds(half + quarter + eighth, eighth)], opp,
                  half + quarter + eighth, eighth)

        for snd in (send_r_a, send_l_a, send_r_b, send_l_b,
                    fwd_r1, fwd_r2, fwd_r3, fwd_l1, fwd_l2, fwd_l3):
            snd.wait_send()

    return pl.pallas_call(
        body,
        out_shape=jax.ShapeDtypeStruct((N_DEV * m_per, n_per), jnp.float32),
        in_specs=[
            pl.BlockSpec(memory_space=pltpu.MemorySpace.HBM),
            pl.BlockSpec(memory_space=pltpu.MemorySpace.HBM),
            pl.BlockSpec(memory_space=pltpu.SMEM),
        ],
        out_specs=pl.BlockSpec(memory_space=pltpu.VMEM),
        scratch_shapes=[
            pltpu.VMEM((m_per, k), x.dtype),
            pltpu.VMEM((k, n_per), w_mat.dtype),
            pltpu.VMEM((m_per, k), x.dtype),
            pltpu.VMEM((m_per, k), x.dtype),
            pltpu.VMEM((m_per, k), x.dtype),
            pltpu.SemaphoreType.DMA((10,)),
            pltpu.SemaphoreType.DMA((10,)),
            pltpu.SemaphoreType.DMA((2,)),
        ],
        compiler_params=pltpu.CompilerParams(collective_id=0),
    )(
        pltpu.with_memory_space_constraint(x, pltpu.MemorySpace.HBM),
        pltpu.with_memory_space_constraint(w_mat, pltpu.MemorySpace.HBM),
        s,
    )
